# Initial kernel scaffold; baseline (speedup 1.0000x reference)
import jax
import jax.numpy as jnp
from jax.experimental import pallas as pl


def kernel(x, edge_index, W0, W1, W2, b0, b1, b2, cb0, cb1, cb2):
    raise NotImplementedError("write your pallas kernel here")


# speedup vs baseline: 4.3672x; 4.3672x over previous
import functools

import jax
import jax.numpy as jnp
from jax import lax
from jax.experimental import pallas as pl
from jax.experimental.pallas import tpu as pltpu
from jax.experimental.pallas import tpu_sc as plsc

N = 10000
D = 256
K = 1024
COMMIT_W = 0.25

NC = 2
NS = 16
CH = 128

NPAD = 10112
E2 = 160000 + N
E2P = 172032
PADE = E2P - E2
AGG_CHUNKS = E2P // NS // CH
DEG_CHUNKS = E2P // (NC * NS) // CH
ZROWS_PER_TILE = NPAD // NS

_sc_mesh = plsc.VectorSubcoreMesh(core_axis_name="c", subcore_axis_name="s")


@functools.partial(
    pl.kernel,
    mesh=_sc_mesh,
    out_type=jax.ShapeDtypeStruct((NC, NPAD), jnp.float32),
    scratch_types=[
        pltpu.VMEM((CH,), jnp.int32),
        pltpu.VMEM((CH,), jnp.float32),
        pltpu.VMEM_SHARED((NPAD,), jnp.float32),
    ],
)
def _deg_sc(dst_hbm, zero_hbm, out_hbm, idx_v, ones_v, acc_sh):
    c = lax.axis_index("c")
    s = lax.axis_index("s")

    @pl.when(s == 0)
    def _():
        pltpu.sync_copy(zero_hbm, acc_sh)

    for j in range(CH // 16):
        ones_v[pl.ds(j * 16, 16)] = jnp.ones((16,), jnp.float32)
    plsc.subcore_barrier()

    wid = c * NS + s

    def body(k, carry):
        base = (wid * DEG_CHUNKS + k) * CH
        pltpu.sync_copy(dst_hbm.at[pl.ds(base, CH)], idx_v)
        pltpu.sync_copy(ones_v, acc_sh.at[idx_v], add=True)
        return carry

    lax.fori_loop(0, DEG_CHUNKS, body, 0)
    plsc.subcore_barrier()

    @pl.when(s == 0)
    def _():
        pltpu.sync_copy(acc_sh, out_hbm.at[c])


@functools.partial(
    pl.kernel,
    mesh=_sc_mesh,
    out_type=[jax.ShapeDtypeStruct((NPAD, 128), jnp.float32)] * 2,
    scratch_types=[
        pltpu.VMEM((CH,), jnp.int32),
        pltpu.VMEM((CH,), jnp.int32),
        pltpu.VMEM((CH, 128), jnp.float32),
        pltpu.VMEM_SHARED((NPAD, 128), jnp.float32),
        pltpu.SemaphoreType.DMA,
    ],
)
def _agg_sc(y0_hbm, y1_hbm, src_hbm, dst_hbm, zero_hbm,
            out0_hbm, out1_hbm, src_v, dst_v, rows_v, acc_sh, sem):
    c = lax.axis_index("c")
    s = lax.axis_index("s")

    z0 = s * ZROWS_PER_TILE
    pltpu.sync_copy(zero_hbm.at[pl.ds(z0, ZROWS_PER_TILE)],
                    acc_sh.at[pl.ds(z0, ZROWS_PER_TILE)])
    plsc.subcore_barrier()

    def run(y_hbm):
        def body(k, carry):
            base = (s * AGG_CHUNKS + k) * CH
            pltpu.sync_copy(src_hbm.at[pl.ds(base, CH)], src_v)
            pltpu.sync_copy(dst_hbm.at[pl.ds(base, CH)], dst_v)
            pltpu.async_copy(y_hbm.at[src_v], rows_v, sem).wait()
            pltpu.sync_copy(rows_v, acc_sh.at[dst_v], add=True)
            return carry
        lax.fori_loop(0, AGG_CHUNKS, body, 0)

    @pl.when(c == 0)
    def _():
        run(y0_hbm)

    @pl.when(c == 1)
    def _():
        run(y1_hbm)

    plsc.subcore_barrier()

    @pl.when(c == 0)
    def _():
        pltpu.sync_copy(acc_sh.at[pl.ds(z0, ZROWS_PER_TILE)],
                        out0_hbm.at[pl.ds(z0, ZROWS_PER_TILE)])

    @pl.when(c == 1)
    def _():
        pltpu.sync_copy(acc_sh.at[pl.ds(z0, ZROWS_PER_TILE)],
                        out1_hbm.at[pl.ds(z0, ZROWS_PER_TILE)])


def _cbnorm_body(cb_ref, out_ref):
    cbl = cb_ref[0]
    nrm = jnp.sqrt(jnp.sum(cbl * cbl, axis=1, keepdims=True)) + 1e-8
    out_ref[0] = cbl / nrm


_cbnorm = pl.pallas_call(
    _cbnorm_body,
    grid=(9,),
    in_specs=[pl.BlockSpec((1, K, D), lambda i: (i, 0, 0))],
    out_specs=pl.BlockSpec((1, K, D), lambda i: (i, 0, 0)),
    out_shape=jax.ShapeDtypeStruct((9, K, D), jnp.float32),
)

BN = 1000


def _xw_body(h_ref, w_ref, deg_ref, y0_ref, y1_ref):
    xw = jnp.dot(h_ref[...], w_ref[...], preferred_element_type=jnp.float32)
    y = xw * lax.rsqrt(deg_ref[...])
    y0_ref[...] = y[:, :128]
    y1_ref[...] = y[:, 128:]


_xw = pl.pallas_call(
    _xw_body,
    grid=(N // BN,),
    in_specs=[
        pl.BlockSpec((BN, D), lambda i: (i, 0)),
        pl.BlockSpec((D, D), lambda i: (0, 0)),
        pl.BlockSpec((BN, 1), lambda i: (i, 0)),
    ],
    out_specs=[pl.BlockSpec((BN, 128), lambda i: (i, 0))] * 2,
    out_shape=[jax.ShapeDtypeStruct((N, 128), jnp.float32)] * 2,
)


def _vq_body(a0_ref, a1_ref, deg_ref, b_ref, h_ref, cbn_ref,
             hn_ref, ids_ref, loss_ref, *, relu):
    i = pl.program_id(0)
    agg = jnp.concatenate([a0_ref[...], a1_ref[...]], axis=1)
    x1 = agg * lax.rsqrt(deg_ref[...]) + b_ref[...]
    if relu:
        x1 = jnp.maximum(x1, 0.0)
    hn_ref[...] = x1 + h_ref[...]

    iota = lax.broadcasted_iota(jnp.int32, (BN, K), 1)
    r = x1
    lsum = jnp.float32(0.0)
    cols = []
    for q in range(3):
        cbq = cbn_ref[q]
        rn = r / (jnp.sqrt(jnp.sum(r * r, axis=1, keepdims=True)) + 1e-8)
        sim = lax.dot_general(rn, cbq, (((1,), (1,)), ((), ())),
                              preferred_element_type=jnp.float32)
        m = jnp.max(sim, axis=1, keepdims=True)
        idxq = jnp.min(jnp.where(sim == m, iota, K), axis=1)
        oh = (iota == idxq[:, None]).astype(jnp.float32)
        qv = lax.dot_general(oh, cbq, (((1,), (0,)), ((), ())),
                             precision=lax.Precision.HIGHEST,
                             preferred_element_type=jnp.float32)
        d = r - qv
        lsum = lsum + jnp.sum(d * d)
        r = d
        cols.append(idxq[:, None])
    ids_ref[...] = jnp.concatenate(cols, axis=1)
    part = lsum * (COMMIT_W / (N * D))

    @pl.when(i == 0)
    def _():
        loss_ref[0, 0] = part

    @pl.when(i > 0)
    def _():
        loss_ref[0, 0] += part


def _make_vq(relu):
    return pl.pallas_call(
        functools.partial(_vq_body, relu=relu),
        grid=(N // BN,),
        in_specs=[
            pl.BlockSpec((BN, 128), lambda i: (i, 0)),
            pl.BlockSpec((BN, 128), lambda i: (i, 0)),
            pl.BlockSpec((BN, 1), lambda i: (i, 0)),
            pl.BlockSpec((1, D), lambda i: (0, 0)),
            pl.BlockSpec((BN, D), lambda i: (i, 0)),
            pl.BlockSpec((3, K, D), lambda i: (0, 0, 0)),
        ],
        out_specs=[
            pl.BlockSpec((BN, D), lambda i: (i, 0)),
            pl.BlockSpec((BN, 3), lambda i: (i, 0)),
            pl.BlockSpec(memory_space=pltpu.MemorySpace.SMEM),
        ],
        out_shape=[
            jax.ShapeDtypeStruct((N, D), jnp.float32),
            jax.ShapeDtypeStruct((N, 3), jnp.int32),
            jax.ShapeDtypeStruct((1, 1), jnp.float32),
        ],
    )


_vq_relu = _make_vq(True)
_vq_last = _make_vq(False)


def kernel(x, edge_index, W0, W1, W2, b0, b1, b2, cb0, cb1, cb2):
    sl = jnp.arange(N, dtype=jnp.int32)
    src2 = jnp.concatenate([edge_index[0].astype(jnp.int32), sl,
                            jnp.zeros((PADE,), jnp.int32)])
    dst2 = jnp.concatenate([edge_index[1].astype(jnp.int32), sl,
                            jnp.full((PADE,), N, jnp.int32)])
    zero1 = jnp.zeros((NPAD,), jnp.float32)
    zero2 = jnp.zeros((NPAD, 128), jnp.float32)

    degp = _deg_sc(dst2, zero1)
    deg = (degp[0] + degp[1])[:N].reshape(N, 1)
    cbn = _cbnorm(jnp.concatenate([cb0, cb1, cb2], axis=0))

    h = x
    loss = jnp.float32(0.0)
    ids_list = []
    for i, (W, b) in enumerate(((W0, b0), (W1, b1), (W2, b2))):
        y0, y1 = _xw(h, W, deg)
        a0, a1 = _agg_sc(y0, y1, src2, dst2, zero2)
        vq = _vq_relu if i < 2 else _vq_last
        h, ids, l = vq(a0, a1, deg, b.reshape(1, D), h, cbn[i * 3:(i + 1) * 3])
        loss = loss + l[0, 0]
        ids_list.append(ids)
    return h, loss, jnp.concatenate(ids_list, axis=1)


# speedup vs baseline: 5.4573x; 1.2496x over previous
import functools

import jax
import jax.numpy as jnp
from jax import lax
from jax.experimental import pallas as pl
from jax.experimental.pallas import tpu as pltpu
from jax.experimental.pallas import tpu_sc as plsc

N = 10000
D = 256
K = 1024
COMMIT_W = 0.25

NC = 2
NS = 16
CH = 128

NPAD = 10112
E2 = 160000 + N
E2P = 172032
PADE = E2P - E2
AGG_CHUNKS = E2P // NS // CH
DEG_CHUNKS = E2P // (NC * NS) // CH
ZROWS_PER_TILE = NPAD // NS

_sc_mesh = plsc.VectorSubcoreMesh(core_axis_name="c", subcore_axis_name="s")


@functools.partial(
    pl.kernel,
    mesh=_sc_mesh,
    out_type=jax.ShapeDtypeStruct((NC, NPAD), jnp.float32),
    scratch_types=[
        pltpu.VMEM((DEG_CHUNKS, CH), jnp.int32),
        pltpu.VMEM((CH,), jnp.float32),
        pltpu.VMEM_SHARED((NPAD,), jnp.float32),
    ],
)
def _deg_sc(dst_hbm, zero_hbm, out_hbm, idx_v, ones_v, acc_sh):
    c = lax.axis_index("c")
    s = lax.axis_index("s")

    @pl.when(s == 0)
    def _():
        pltpu.sync_copy(zero_hbm, acc_sh)

    wid = c * NS + s
    pltpu.sync_copy(dst_hbm.at[wid], idx_v)
    for j in range(CH // 16):
        ones_v[pl.ds(j * 16, 16)] = jnp.ones((16,), jnp.float32)
    plsc.subcore_barrier()

    def body(k, carry):
        pltpu.sync_copy(ones_v, acc_sh.at[idx_v.at[k]], add=True)
        return carry

    lax.fori_loop(0, DEG_CHUNKS, body, 0)
    plsc.subcore_barrier()

    @pl.when(s == 0)
    def _():
        pltpu.sync_copy(acc_sh, out_hbm.at[c])


NBUF = 2
HALF = AGG_CHUNKS // 2
NGROUPS = HALF // NBUF


@functools.partial(
    pl.kernel,
    mesh=_sc_mesh,
    out_type=[jax.ShapeDtypeStruct((NPAD, 128), jnp.float32)] * 2,
    scratch_types=[
        pltpu.VMEM((HALF, CH), jnp.int32),
        pltpu.VMEM((HALF, CH), jnp.int32),
    ] + [pltpu.VMEM((CH, 128), jnp.float32)] * NBUF
      + [pltpu.SemaphoreType.DMA] * NBUF
      + [pltpu.VMEM_SHARED((NPAD, 128), jnp.float32)],
)
def _agg_sc(y0_hbm, y1_hbm, src_hbm, dst_hbm, zero_hbm,
            out0_hbm, out1_hbm, src_v, dst_v,
            rb0, rb1, sm0, sm1, acc_sh):
    c = lax.axis_index("c")
    s = lax.axis_index("s")
    rows = [rb0, rb1]
    sems = [sm0, sm1]

    z0 = s * ZROWS_PER_TILE
    pltpu.sync_copy(zero_hbm.at[pl.ds(z0, ZROWS_PER_TILE)],
                    acc_sh.at[pl.ds(z0, ZROWS_PER_TILE)])
    plsc.subcore_barrier()

    def run(y_hbm):
        for h in range(2):
            pltpu.sync_copy(src_hbm.at[s * 2 + h], src_v)
            pltpu.sync_copy(dst_hbm.at[s * 2 + h], dst_v)
            for j in range(NBUF):
                pltpu.make_async_copy(y_hbm.at[src_v.at[j]], rows[j], sems[j]).start()

            def group(g, carry):
                for j in range(NBUF):
                    k = g * NBUF + j
                    pltpu.make_async_copy(y_hbm.at[src_v.at[k]],
                                          rows[j], sems[j]).wait()
                    pltpu.sync_copy(rows[j], acc_sh.at[dst_v.at[k]], add=True)

                    @pl.when(g < NGROUPS - 1)
                    def _():
                        pltpu.make_async_copy(y_hbm.at[src_v.at[k + NBUF]],
                                              rows[j], sems[j]).start()
                return carry

            lax.fori_loop(0, NGROUPS, group, 0)

    @pl.when(c == 0)
    def _():
        run(y0_hbm)

    @pl.when(c == 1)
    def _():
        run(y1_hbm)

    plsc.subcore_barrier()

    @pl.when(c == 0)
    def _():
        pltpu.sync_copy(acc_sh.at[pl.ds(z0, ZROWS_PER_TILE)],
                        out0_hbm.at[pl.ds(z0, ZROWS_PER_TILE)])

    @pl.when(c == 1)
    def _():
        pltpu.sync_copy(acc_sh.at[pl.ds(z0, ZROWS_PER_TILE)],
                        out1_hbm.at[pl.ds(z0, ZROWS_PER_TILE)])


def _cbnorm_body(cb_ref, out_ref):
    cbl = cb_ref[0]
    nrm = jnp.sqrt(jnp.sum(cbl * cbl, axis=1, keepdims=True)) + 1e-8
    out_ref[0] = cbl / nrm


_cbnorm = pl.pallas_call(
    _cbnorm_body,
    grid=(9,),
    in_specs=[pl.BlockSpec((1, K, D), lambda i: (i, 0, 0))],
    out_specs=pl.BlockSpec((1, K, D), lambda i: (i, 0, 0)),
    out_shape=jax.ShapeDtypeStruct((9, K, D), jnp.float32),
)

BN = 1000


def _xw_body(h_ref, w_ref, deg_ref, y0_ref, y1_ref):
    xw = jnp.dot(h_ref[...], w_ref[...], preferred_element_type=jnp.float32)
    y = xw * lax.rsqrt(deg_ref[...])
    y0_ref[...] = y[:, :128]
    y1_ref[...] = y[:, 128:]


_xw = pl.pallas_call(
    _xw_body,
    grid=(N // BN,),
    in_specs=[
        pl.BlockSpec((BN, D), lambda i: (i, 0)),
        pl.BlockSpec((D, D), lambda i: (0, 0)),
        pl.BlockSpec((BN, 1), lambda i: (i, 0)),
    ],
    out_specs=[pl.BlockSpec((BN, 128), lambda i: (i, 0))] * 2,
    out_shape=[jax.ShapeDtypeStruct((N, 128), jnp.float32)] * 2,
)


def _vq_body(a0_ref, a1_ref, deg_ref, b_ref, h_ref, cbn_ref,
             hn_ref, ids_ref, loss_ref, *, relu):
    i = pl.program_id(0)
    agg = jnp.concatenate([a0_ref[...], a1_ref[...]], axis=1)
    x1 = agg * lax.rsqrt(deg_ref[...]) + b_ref[...]
    if relu:
        x1 = jnp.maximum(x1, 0.0)
    hn_ref[...] = x1 + h_ref[...]

    iota = lax.broadcasted_iota(jnp.int32, (BN, K), 1)
    r = x1
    lsum = jnp.float32(0.0)
    cols = []
    for q in range(3):
        cbq = cbn_ref[q]
        rn = r / (jnp.sqrt(jnp.sum(r * r, axis=1, keepdims=True)) + 1e-8)
        sim = lax.dot_general(rn, cbq, (((1,), (1,)), ((), ())),
                              preferred_element_type=jnp.float32)
        m = jnp.max(sim, axis=1, keepdims=True)
        idxq = jnp.min(jnp.where(sim == m, iota, K), axis=1)
        oh = (iota == idxq[:, None]).astype(jnp.float32)
        qv = lax.dot_general(oh, cbq, (((1,), (0,)), ((), ())),
                             precision=lax.Precision.HIGHEST,
                             preferred_element_type=jnp.float32)
        d = r - qv
        lsum = lsum + jnp.sum(d * d)
        r = d
        cols.append(idxq[:, None])
    ids_ref[...] = jnp.concatenate(cols, axis=1)
    part = lsum * (COMMIT_W / (N * D))

    @pl.when(i == 0)
    def _():
        loss_ref[0, 0] = part

    @pl.when(i > 0)
    def _():
        loss_ref[0, 0] += part


def _make_vq(relu):
    return pl.pallas_call(
        functools.partial(_vq_body, relu=relu),
        grid=(N // BN,),
        in_specs=[
            pl.BlockSpec((BN, 128), lambda i: (i, 0)),
            pl.BlockSpec((BN, 128), lambda i: (i, 0)),
            pl.BlockSpec((BN, 1), lambda i: (i, 0)),
            pl.BlockSpec((1, D), lambda i: (0, 0)),
            pl.BlockSpec((BN, D), lambda i: (i, 0)),
            pl.BlockSpec((3, K, D), lambda i: (0, 0, 0)),
        ],
        out_specs=[
            pl.BlockSpec((BN, D), lambda i: (i, 0)),
            pl.BlockSpec((BN, 3), lambda i: (i, 0)),
            pl.BlockSpec(memory_space=pltpu.MemorySpace.SMEM),
        ],
        out_shape=[
            jax.ShapeDtypeStruct((N, D), jnp.float32),
            jax.ShapeDtypeStruct((N, 3), jnp.int32),
            jax.ShapeDtypeStruct((1, 1), jnp.float32),
        ],
    )


_vq_relu = _make_vq(True)
_vq_last = _make_vq(False)


def kernel(x, edge_index, W0, W1, W2, b0, b1, b2, cb0, cb1, cb2):
    sl = jnp.arange(N, dtype=jnp.int32)
    src2 = jnp.concatenate([edge_index[0].astype(jnp.int32), sl,
                            jnp.zeros((PADE,), jnp.int32)])
    dst2 = jnp.concatenate([edge_index[1].astype(jnp.int32), sl,
                            jnp.full((PADE,), N, jnp.int32)])
    src3 = src2.reshape(NS * 2, HALF, CH)
    dst3 = dst2.reshape(NS * 2, HALF, CH)
    dst4 = dst2.reshape(NC * NS, DEG_CHUNKS, CH)
    zero1 = jnp.zeros((NPAD,), jnp.float32)
    zero2 = jnp.zeros((NPAD, 128), jnp.float32)

    degp = _deg_sc(dst4, zero1)
    deg = (degp[0] + degp[1])[:N].reshape(N, 1)
    cbn = _cbnorm(jnp.concatenate([cb0, cb1, cb2], axis=0))

    h = x
    loss = jnp.float32(0.0)
    ids_list = []
    for i, (W, b) in enumerate(((W0, b0), (W1, b1), (W2, b2))):
        y0, y1 = _xw(h, W, deg)
        a0, a1 = _agg_sc(y0, y1, src3, dst3, zero2)
        vq = _vq_relu if i < 2 else _vq_last
        h, ids, l = vq(a0, a1, deg, b.reshape(1, D), h, cbn[i * 3:(i + 1) * 3])
        loss = loss + l[0, 0]
        ids_list.append(ids)
    return h, loss, jnp.concatenate(ids_list, axis=1)
